# Initial kernel scaffold; baseline (speedup 1.0000x reference)
import functools

import jax
import jax.numpy as jnp
from jax import lax
from jax.experimental import pallas as pl
from jax.experimental.pallas import tpu as pltpu
from jax.experimental.pallas import tpu_sc as plsc

_N = 100000
_E = 3200000
_CH = 128
_GROUP = 8
_NCHUNKS = _E // _CH
_NGROUPS = _E // (_CH * _GROUP)
_NW = 32
_GROUPS_PER_W = -(-_NGROUPS // _NW)
_RPT = _N // 16
_ZROWS = 1250
_CW = 6256
_CW_LAST = _N - 15 * _CW
_BN = 2000
_GRID = _N // _BN


def _sc_agg_body(src_hbm, dst_hbm, h_hbm, ssum_out, cnt_out,
                 sidx, didx, rbuf, ones, zrows, zwords,
                 ssum_sh, cnt_sh, sem0, sem1):
    cid = lax.axis_index("c")
    sid = lax.axis_index("s")
    wid = cid * 16 + sid

    zero16 = jnp.zeros((16,), jnp.float32)
    one16 = jnp.ones((16,), jnp.float32)

    def zfill(i, c):
        zrows[i] = zero16
        return c
    lax.fori_loop(0, _ZROWS, zfill, 0)

    def zfillw(i, c):
        zwords[pl.ds(i * 16, 16)] = zero16
        return c
    lax.fori_loop(0, _CW // 16, zfillw, 0)

    for j in range(_CH // 16):
        ones[pl.ds(j * 16, 16)] = one16

    rbase = sid * _RPT
    for b in range(5):
        pltpu.sync_copy(zrows, ssum_sh.at[pl.ds(rbase + b * _ZROWS, _ZROWS)])

    @pl.when(sid < 15)
    def _():
        pltpu.sync_copy(zwords, cnt_sh.at[pl.ds(sid * _CW, _CW)])

    @pl.when(sid == 15)
    def _():
        pltpu.sync_copy(zwords.at[pl.ds(0, _CW_LAST)],
                        cnt_sh.at[pl.ds(15 * _CW, _CW_LAST)])

    plsc.subcore_barrier()

    def group_body(k, c):
        g = wid + k * _NW

        @pl.when(g < _NGROUPS)
        def _():
            pltpu.sync_copy(src_hbm.at[pl.ds(g * _GROUP, _GROUP)], sidx)
            pltpu.sync_copy(dst_hbm.at[pl.ds(g * _GROUP, _GROUP)], didx)
            d_prev = pltpu.async_copy(h_hbm.at[sidx.at[0]], rbuf.at[0], sem0)
            for j in range(_GROUP):
                d_next = None
                if j + 1 < _GROUP:
                    d_next = pltpu.async_copy(
                        h_hbm.at[sidx.at[j + 1]], rbuf.at[(j + 1) % 2],
                        sem1 if (j + 1) % 2 else sem0)
                d_prev.wait()
                pltpu.sync_copy(rbuf.at[j % 2], ssum_sh.at[didx.at[j]], add=True)
                pltpu.sync_copy(ones, cnt_sh.at[didx.at[j]], add=True)
                d_prev = d_next
        return c
    lax.fori_loop(0, _GROUPS_PER_W, group_body, 0)

    plsc.subcore_barrier()

    orow = cid * _N + rbase
    for b in range(5):
        pltpu.sync_copy(ssum_sh.at[pl.ds(rbase + b * _ZROWS, _ZROWS)], zrows)
        pltpu.sync_copy(zrows, ssum_out.at[pl.ds(orow + b * _ZROWS, _ZROWS)])

    @pl.when(sid < 15)
    def _():
        pltpu.sync_copy(cnt_sh.at[pl.ds(sid * _CW, _CW)], zwords)
        pltpu.sync_copy(zwords, cnt_out.at[pl.ds(cid * _N + sid * _CW, _CW)])

    @pl.when(sid == 15)
    def _():
        pltpu.sync_copy(cnt_sh.at[pl.ds(15 * _CW, _CW_LAST)],
                        zwords.at[pl.ds(0, _CW_LAST)])
        pltpu.sync_copy(zwords.at[pl.ds(0, _CW_LAST)],
                        cnt_out.at[pl.ds(cid * _N + 15 * _CW, _CW_LAST)])


_sc_agg = functools.partial(
    pl.kernel,
    out_type=(jax.ShapeDtypeStruct((2 * _N, 16), jnp.float32),
              jax.ShapeDtypeStruct((2 * _N,), jnp.float32)),
    mesh=plsc.VectorSubcoreMesh(core_axis_name="c", subcore_axis_name="s"),
    scratch_types=[
        pltpu.VMEM((_GROUP, _CH), jnp.int32),
        pltpu.VMEM((_GROUP, _CH), jnp.int32),
        pltpu.VMEM((2, _CH, 16), jnp.float32),
        pltpu.VMEM((_CH,), jnp.float32),
        pltpu.VMEM((_ZROWS, 16), jnp.float32),
        pltpu.VMEM((_CW,), jnp.float32),
        pltpu.VMEM_SHARED((_N, 16), jnp.float32),
        pltpu.VMEM_SHARED((_N,), jnp.float32),
        pltpu.SemaphoreType.DMA,
        pltpu.SemaphoreType.DMA,
    ],
)(_sc_agg_body)


def _lin1_body(x_ref, w_ref, b_ref, o_ref):
    o_ref[...] = jnp.maximum(
        jnp.dot(x_ref[...], w_ref[...], preferred_element_type=jnp.float32)
        + b_ref[...], 0.0)


def _final_body(s0, s1, c0, c1, h_ref, wl, bl, wr, w2, b2, o_ref):
    cnt = jnp.maximum(c0[...] + c1[...], 1.0)
    aggr = (s0[...] + s1[...]) / cnt
    hh = h_ref[...]
    h2 = jnp.maximum(
        jnp.dot(aggr, wl[...], preferred_element_type=jnp.float32) + bl[...]
        + jnp.dot(hh, wr[...], preferred_element_type=jnp.float32), 0.0)
    o_ref[...] = jnp.dot(h2, w2[...], preferred_element_type=jnp.float32) + b2[...]


@jax.jit
def kernel(x, edge_index, W1, b1, Wl, bl, Wr, W2, b2):
    h = pl.pallas_call(
        _lin1_body,
        grid=(_GRID,),
        in_specs=[pl.BlockSpec((_BN, 16), lambda i: (i, 0)),
                  pl.BlockSpec((16, 16), lambda i: (0, 0)),
                  pl.BlockSpec((1, 16), lambda i: (0, 0))],
        out_specs=pl.BlockSpec((_BN, 16), lambda i: (i, 0)),
        out_shape=jax.ShapeDtypeStruct((_N, 16), jnp.float32),
    )(x, W1.T, b1.reshape(1, 16))

    src2d = edge_index[0].reshape(_NCHUNKS, _CH)
    dst2d = edge_index[1].reshape(_NCHUNKS, _CH)
    ssum_p, cnt_p = _sc_agg(src2d, dst2d, h)
    cnt2d = cnt_p.reshape(2 * _N, 1)

    out = pl.pallas_call(
        _final_body,
        grid=(_GRID,),
        in_specs=[pl.BlockSpec((_BN, 16), lambda i: (i, 0)),
                  pl.BlockSpec((_BN, 16), lambda i: (i + _GRID, 0)),
                  pl.BlockSpec((_BN, 1), lambda i: (i, 0)),
                  pl.BlockSpec((_BN, 1), lambda i: (i + _GRID, 0)),
                  pl.BlockSpec((_BN, 16), lambda i: (i, 0)),
                  pl.BlockSpec((16, 16), lambda i: (0, 0)),
                  pl.BlockSpec((1, 16), lambda i: (0, 0)),
                  pl.BlockSpec((16, 16), lambda i: (0, 0)),
                  pl.BlockSpec((16, 32), lambda i: (0, 0)),
                  pl.BlockSpec((1, 32), lambda i: (0, 0))],
        out_specs=pl.BlockSpec((_BN, 32), lambda i: (i, 0)),
        out_shape=jax.ShapeDtypeStruct((_N, 32), jnp.float32),
    )(ssum_p, ssum_p, cnt2d, cnt2d, h,
      Wl.T, bl.reshape(1, 16), Wr.T, W2.T, b2.reshape(1, 32))
    return out


# speedup vs baseline: 26.5119x; 26.5119x over previous
import functools

import jax
import jax.numpy as jnp
from jax import lax
from jax.experimental import pallas as pl
from jax.experimental.pallas import tpu as pltpu
from jax.experimental.pallas import tpu_sc as plsc

_N = 100000
_E = 3200000
_CH = 128
_GROUP = 8
_NCHUNKS = _E // _CH
_NGROUPS = _E // (_CH * _GROUP)
_NW = 32
_GROUPS_PER_W = -(-_NGROUPS // _NW)
_CW = 6256
_CW_LAST = _N - 15 * _CW
_ZROWS = _CW // 2
_ZROWS_LAST = _CW_LAST // 2
_BN = 2000
_GRID = _N // _BN


def _sc_agg_body(src_hbm, dst_hbm, h_hbm, z16_hbm, z1_hbm, ssum_out, cnt_out,
                 sidx, didx, rbuf, ones,
                 ssum_sh, cnt_sh, sem0, sem1):
    cid = lax.axis_index("c")
    sid = lax.axis_index("s")
    wid = cid * 16 + sid

    one16 = jnp.ones((16,), jnp.float32)
    for j in range(_CH // 16):
        ones[pl.ds(j * 16, 16)] = one16

    rbase = sid * _CW

    @pl.when(sid < 15)
    def _():
        pltpu.sync_copy(z16_hbm, ssum_sh.at[pl.ds(rbase, _CW)])
        pltpu.sync_copy(z1_hbm, cnt_sh.at[pl.ds(rbase, _CW)])

    @pl.when(sid == 15)
    def _():
        pltpu.sync_copy(z16_hbm.at[pl.ds(0, _CW_LAST)],
                        ssum_sh.at[pl.ds(15 * _CW, _CW_LAST)])
        pltpu.sync_copy(z1_hbm.at[pl.ds(0, _CW_LAST)],
                        cnt_sh.at[pl.ds(15 * _CW, _CW_LAST)])

    plsc.subcore_barrier()

    def group_body(k, c):
        g = wid + k * _NW

        @pl.when(g < _NGROUPS)
        def _():
            pltpu.sync_copy(src_hbm.at[pl.ds(g * _GROUP, _GROUP)], sidx)
            pltpu.sync_copy(dst_hbm.at[pl.ds(g * _GROUP, _GROUP)], didx)
            d_prev = pltpu.async_copy(h_hbm.at[sidx.at[0]], rbuf.at[0], sem0)
            for j in range(_GROUP):
                d_next = None
                if j + 1 < _GROUP:
                    d_next = pltpu.async_copy(
                        h_hbm.at[sidx.at[j + 1]], rbuf.at[(j + 1) % 2],
                        sem1 if (j + 1) % 2 else sem0)
                d_prev.wait()
                pltpu.sync_copy(rbuf.at[j % 2], ssum_sh.at[didx.at[j]], add=True)
                pltpu.sync_copy(ones, cnt_sh.at[didx.at[j]], add=True)
                d_prev = d_next
        return c
    lax.fori_loop(0, _GROUPS_PER_W, group_body, 0)

    plsc.subcore_barrier()

    orow = cid * _N + rbase

    @pl.when(sid < 15)
    def _():
        pltpu.sync_copy(ssum_sh.at[pl.ds(rbase, _CW)],
                        ssum_out.at[pl.ds(orow, _CW)])
        pltpu.sync_copy(cnt_sh.at[pl.ds(rbase, _CW)],
                        cnt_out.at[pl.ds(orow, _CW)])

    @pl.when(sid == 15)
    def _():
        pltpu.sync_copy(ssum_sh.at[pl.ds(15 * _CW, _CW_LAST)],
                        ssum_out.at[pl.ds(cid * _N + 15 * _CW, _CW_LAST)])
        pltpu.sync_copy(cnt_sh.at[pl.ds(15 * _CW, _CW_LAST)],
                        cnt_out.at[pl.ds(cid * _N + 15 * _CW, _CW_LAST)])


_sc_agg = functools.partial(
    pl.kernel,
    out_type=(jax.ShapeDtypeStruct((2 * _N, 16), jnp.float32),
              jax.ShapeDtypeStruct((2 * _N,), jnp.float32)),
    mesh=plsc.VectorSubcoreMesh(core_axis_name="c", subcore_axis_name="s"),
    compiler_params=pltpu.CompilerParams(use_tc_tiling_on_sc=False),
    scratch_types=[
        pltpu.VMEM((_GROUP, _CH), jnp.int32),
        pltpu.VMEM((_GROUP, _CH), jnp.int32),
        pltpu.VMEM((2, _CH, 16), jnp.float32),
        pltpu.VMEM((_CH,), jnp.float32),
        pltpu.VMEM_SHARED((_N, 16), jnp.float32),
        pltpu.VMEM_SHARED((_N,), jnp.float32),
        pltpu.SemaphoreType.DMA,
        pltpu.SemaphoreType.DMA,
    ],
)(_sc_agg_body)


def _lin1_body(x_ref, w_ref, b_ref, o_ref):
    o_ref[...] = jnp.maximum(
        jnp.dot(x_ref[...], w_ref[...], preferred_element_type=jnp.float32)
        + b_ref[...], 0.0)


def _final_body(s0, s1, c0, c1, h_ref, wl, bl, wr, w2, b2, o_ref):
    cnt = jnp.maximum(c0[...] + c1[...], 1.0)
    aggr = (s0[...] + s1[...]) / cnt
    hh = h_ref[...]
    h2 = jnp.maximum(
        jnp.dot(aggr, wl[...], preferred_element_type=jnp.float32) + bl[...]
        + jnp.dot(hh, wr[...], preferred_element_type=jnp.float32), 0.0)
    o_ref[...] = jnp.dot(h2, w2[...], preferred_element_type=jnp.float32) + b2[...]


@jax.jit
def kernel(x, edge_index, W1, b1, Wl, bl, Wr, W2, b2):
    h = pl.pallas_call(
        _lin1_body,
        grid=(_GRID,),
        in_specs=[pl.BlockSpec((_BN, 16), lambda i: (i, 0)),
                  pl.BlockSpec((16, 16), lambda i: (0, 0)),
                  pl.BlockSpec((1, 16), lambda i: (0, 0))],
        out_specs=pl.BlockSpec((_BN, 16), lambda i: (i, 0)),
        out_shape=jax.ShapeDtypeStruct((_N, 16), jnp.float32),
    )(x, W1.T, b1.reshape(1, 16))

    src2d = edge_index[0].reshape(_NCHUNKS, _CH)
    dst2d = edge_index[1].reshape(_NCHUNKS, _CH)
    z16 = jnp.zeros((_CW, 16), jnp.float32)
    z1 = jnp.zeros((_CW,), jnp.float32)
    ssum_p, cnt_p = _sc_agg(src2d, dst2d, h, z16, z1)
    cnt2d = cnt_p.reshape(2 * _N, 1)

    out = pl.pallas_call(
        _final_body,
        grid=(_GRID,),
        in_specs=[pl.BlockSpec((_BN, 16), lambda i: (i, 0)),
                  pl.BlockSpec((_BN, 16), lambda i: (i + _GRID, 0)),
                  pl.BlockSpec((_BN, 1), lambda i: (i, 0)),
                  pl.BlockSpec((_BN, 1), lambda i: (i + _GRID, 0)),
                  pl.BlockSpec((_BN, 16), lambda i: (i, 0)),
                  pl.BlockSpec((16, 16), lambda i: (0, 0)),
                  pl.BlockSpec((1, 16), lambda i: (0, 0)),
                  pl.BlockSpec((16, 16), lambda i: (0, 0)),
                  pl.BlockSpec((16, 32), lambda i: (0, 0)),
                  pl.BlockSpec((1, 32), lambda i: (0, 0))],
        out_specs=pl.BlockSpec((_BN, 32), lambda i: (i, 0)),
        out_shape=jax.ShapeDtypeStruct((_N, 32), jnp.float32),
    )(ssum_p, ssum_p, cnt2d, cnt2d, h,
      Wl.T, bl.reshape(1, 16), Wr.T, W2.T, b2.reshape(1, 32))
    return out


# speedup vs baseline: 36.0886x; 1.3612x over previous
import functools

import jax
import jax.numpy as jnp
from jax import lax
from jax.experimental import pallas as pl
from jax.experimental.pallas import tpu as pltpu
from jax.experimental.pallas import tpu_sc as plsc

_N = 100000
_E = 3200000
_CH = 128
_GROUP = 8
_NW = 32
_GP = 98
_EP = _NW * _GP * _CH * _GROUP
_NCHUNKS = _EP // _CH
_NA = _N + 96
_CW = 6256
_CW_LAST = _N - 15 * _CW
_BN = 2000
_GRID = _N // _BN


def _sc_agg_body(src_hbm, dst_hbm, h_hbm, z16_hbm, z1_hbm, ssum_out, cnt_out,
                 sidx, didx, rbuf, ones,
                 ssum_sh, cnt_sh,
                 g0, g1, g2, g3, g4, g5, g6, g7, ssem, csem, isem):
    cid = lax.axis_index("c")
    sid = lax.axis_index("s")
    wid = cid * 16 + sid
    gsems = (g0, g1, g2, g3, g4, g5, g6, g7)

    one16 = jnp.ones((16,), jnp.float32)
    for j in range(_CH // 16):
        ones[pl.ds(j * 16, 16)] = one16

    rbase = sid * _CW
    pltpu.sync_copy(z16_hbm, ssum_sh.at[pl.ds(rbase, _CW)])
    pltpu.sync_copy(z1_hbm, cnt_sh.at[pl.ds(rbase, _CW)])

    plsc.subcore_barrier()

    def fire_idx(b, g):
        pltpu.async_copy(src_hbm.at[pl.ds(g * _GROUP, _GROUP)], sidx.at[b],
                         isem)
        pltpu.async_copy(dst_hbm.at[pl.ds(g * _GROUP, _GROUP)], didx.at[b],
                         isem)

    def wait_idx(b):
        pltpu.make_async_copy(src_hbm.at[pl.ds(0, _GROUP)], sidx.at[b],
                              isem).wait()
        pltpu.make_async_copy(dst_hbm.at[pl.ds(0, _GROUP)], didx.at[b],
                              isem).wait()

    def drain_scatters(ob):
        for j in range(_GROUP):
            pltpu.make_async_copy(rbuf.at[j], ssum_sh.at[didx.at[ob, j]],
                                  ssem).wait()
            pltpu.make_async_copy(ones, cnt_sh.at[didx.at[ob, j]],
                                  csem).wait()

    def run_group(b):
        gd = [pltpu.async_copy(h_hbm.at[sidx.at[b, j]], rbuf.at[j], gsems[j])
              for j in range(_GROUP)]
        for j in range(_GROUP):
            gd[j].wait()
            pltpu.async_copy(rbuf.at[j], ssum_sh.at[didx.at[b, j]], ssem,
                             add=True)
            pltpu.async_copy(ones, cnt_sh.at[didx.at[b, j]], csem, add=True)

    fire_idx(0, wid)

    def pair_body(k2, c):
        k0 = 2 * k2

        @pl.when(k2 > 0)
        def _():
            drain_scatters(1)
        wait_idx(0)
        fire_idx(1, wid + (k0 + 1) * _NW)
        run_group(0)

        drain_scatters(0)
        wait_idx(1)

        @pl.when(k2 < _GP // 2 - 1)
        def _():
            fire_idx(0, wid + (k0 + 2) * _NW)
        run_group(1)
        return c
    lax.fori_loop(0, _GP // 2, pair_body, 0)
    drain_scatters(1)

    plsc.subcore_barrier()

    orow = cid * _N + rbase

    @pl.when(sid < 15)
    def _():
        pltpu.sync_copy(ssum_sh.at[pl.ds(rbase, _CW)],
                        ssum_out.at[pl.ds(orow, _CW)])
        pltpu.sync_copy(cnt_sh.at[pl.ds(rbase, _CW)],
                        cnt_out.at[pl.ds(orow, _CW)])

    @pl.when(sid == 15)
    def _():
        pltpu.sync_copy(ssum_sh.at[pl.ds(15 * _CW, _CW_LAST)],
                        ssum_out.at[pl.ds(cid * _N + 15 * _CW, _CW_LAST)])
        pltpu.sync_copy(cnt_sh.at[pl.ds(15 * _CW, _CW_LAST)],
                        cnt_out.at[pl.ds(cid * _N + 15 * _CW, _CW_LAST)])


_sc_agg = functools.partial(
    pl.kernel,
    out_type=(jax.ShapeDtypeStruct((2 * _N, 16), jnp.float32),
              jax.ShapeDtypeStruct((2 * _N,), jnp.float32)),
    mesh=plsc.VectorSubcoreMesh(core_axis_name="c", subcore_axis_name="s"),
    compiler_params=pltpu.CompilerParams(use_tc_tiling_on_sc=False),
    scratch_types=[
        pltpu.VMEM((2, _GROUP, _CH), jnp.int32),
        pltpu.VMEM((2, _GROUP, _CH), jnp.int32),
        pltpu.VMEM((_GROUP, _CH, 16), jnp.float32),
        pltpu.VMEM((_CH,), jnp.float32),
        pltpu.VMEM_SHARED((_NA, 16), jnp.float32),
        pltpu.VMEM_SHARED((_NA,), jnp.float32),
    ] + [pltpu.SemaphoreType.DMA] * 11,
)(_sc_agg_body)


def _lin1_body(x_ref, w_ref, b_ref, o_ref):
    o_ref[...] = jnp.maximum(
        jnp.dot(x_ref[...], w_ref[...], preferred_element_type=jnp.float32)
        + b_ref[...], 0.0)


def _final_body(s0, s1, c0, c1, h_ref, wl, bl, wr, w2, b2, o_ref):
    cnt = jnp.maximum(c0[...] + c1[...], 1.0)
    aggr = (s0[...] + s1[...]) / cnt
    hh = h_ref[...]
    h2 = jnp.maximum(
        jnp.dot(aggr, wl[...], preferred_element_type=jnp.float32) + bl[...]
        + jnp.dot(hh, wr[...], preferred_element_type=jnp.float32), 0.0)
    o_ref[...] = jnp.dot(h2, w2[...], preferred_element_type=jnp.float32) + b2[...]


@jax.jit
def kernel(x, edge_index, W1, b1, Wl, bl, Wr, W2, b2):
    h = pl.pallas_call(
        _lin1_body,
        grid=(_GRID,),
        in_specs=[pl.BlockSpec((_BN, 16), lambda i: (i, 0)),
                  pl.BlockSpec((16, 16), lambda i: (0, 0)),
                  pl.BlockSpec((1, 16), lambda i: (0, 0))],
        out_specs=pl.BlockSpec((_BN, 16), lambda i: (i, 0)),
        out_shape=jax.ShapeDtypeStruct((_N, 16), jnp.float32),
    )(x, W1.T, b1.reshape(1, 16))

    pad_s = jnp.zeros((_EP - _E,), jnp.int32)
    pad_d = jnp.full((_EP - _E,), _N, jnp.int32)
    src2d = jnp.concatenate([edge_index[0], pad_s]).reshape(_NCHUNKS, _CH)
    dst2d = jnp.concatenate([edge_index[1], pad_d]).reshape(_NCHUNKS, _CH)
    z16 = jnp.zeros((_CW, 16), jnp.float32)
    z1 = jnp.zeros((_CW,), jnp.float32)
    ssum_p, cnt_p = _sc_agg(src2d, dst2d, h, z16, z1)
    cnt2d = cnt_p.reshape(2 * _N, 1)

    out = pl.pallas_call(
        _final_body,
        grid=(_GRID,),
        in_specs=[pl.BlockSpec((_BN, 16), lambda i: (i, 0)),
                  pl.BlockSpec((_BN, 16), lambda i: (i + _GRID, 0)),
                  pl.BlockSpec((_BN, 1), lambda i: (i, 0)),
                  pl.BlockSpec((_BN, 1), lambda i: (i + _GRID, 0)),
                  pl.BlockSpec((_BN, 16), lambda i: (i, 0)),
                  pl.BlockSpec((16, 16), lambda i: (0, 0)),
                  pl.BlockSpec((1, 16), lambda i: (0, 0)),
                  pl.BlockSpec((16, 16), lambda i: (0, 0)),
                  pl.BlockSpec((16, 32), lambda i: (0, 0)),
                  pl.BlockSpec((1, 32), lambda i: (0, 0))],
        out_specs=pl.BlockSpec((_BN, 32), lambda i: (i, 0)),
        out_shape=jax.ShapeDtypeStruct((_N, 32), jnp.float32),
    )(ssum_p, ssum_p, cnt2d, cnt2d, h,
      Wl.T, bl.reshape(1, 16), Wr.T, W2.T, b2.reshape(1, 32))
    return out


# speedup vs baseline: 56.6796x; 1.5706x over previous
import functools

import jax
import jax.numpy as jnp
from jax import lax
from jax.experimental import pallas as pl
from jax.experimental.pallas import tpu as pltpu
from jax.experimental.pallas import tpu_sc as plsc

_N = 100000
_E = 3200000
_CH = 128
_GROUP = 8
_NW = 32
_NGROUPS = _E // (_CH * _GROUP)
_NCHUNKS = _E // _CH
_NA = _N + 96
_CW = 6256
_NP = _NA // 8
_BP = 3128
_GRIDP = _NP // _BP


def _sc_agg_body(e3_hbm, h_hbm, z16_hbm, z1_hbm, ssum_out, cnt_out,
                 sidx, didx, rbuf, ones,
                 ssum_sh, cnt_sh,
                 g0, g1, g2, g3, g4, g5, g6, g7, ssem, csem, isem):
    cid = lax.axis_index("c")
    sid = lax.axis_index("s")
    wid = cid * 16 + sid
    gsems = (g0, g1, g2, g3, g4, g5, g6, g7)

    one16 = jnp.ones((16,), jnp.float32)
    for j in range(_CH // 16):
        ones[pl.ds(j * 16, 16)] = one16

    rbase = sid * _CW
    pltpu.sync_copy(z16_hbm, ssum_sh.at[pl.ds(rbase, _CW)])
    pltpu.sync_copy(z1_hbm, cnt_sh.at[pl.ds(rbase, _CW)])

    plsc.subcore_barrier()

    def fire_idx(b, g):
        pltpu.async_copy(e3_hbm.at[0, pl.ds(g * _GROUP, _GROUP)], sidx.at[b],
                         isem)
        pltpu.async_copy(e3_hbm.at[1, pl.ds(g * _GROUP, _GROUP)], didx.at[b],
                         isem)

    def wait_idx(b):
        pltpu.make_async_copy(e3_hbm.at[0, pl.ds(0, _GROUP)], sidx.at[b],
                              isem).wait()
        pltpu.make_async_copy(e3_hbm.at[1, pl.ds(0, _GROUP)], didx.at[b],
                              isem).wait()

    def drain_scatters(ob):
        for j in range(_GROUP):
            pltpu.make_async_copy(rbuf.at[j], ssum_sh.at[didx.at[ob, j]],
                                  ssem).wait()
            pltpu.make_async_copy(ones, cnt_sh.at[didx.at[ob, j]],
                                  csem).wait()

    def run_group(b):
        gd = [pltpu.async_copy(h_hbm.at[sidx.at[b, j]], rbuf.at[j], gsems[j])
              for j in range(_GROUP)]
        for j in range(_GROUP):
            gd[j].wait()
            pltpu.async_copy(rbuf.at[j], ssum_sh.at[didx.at[b, j]], ssem,
                             add=True)
            pltpu.async_copy(ones, cnt_sh.at[didx.at[b, j]], csem, add=True)

    fire_idx(0, wid)

    def pair_body(k2, c):
        k0 = 2 * k2

        @pl.when(k2 > 0)
        def _():
            drain_scatters(1)
        wait_idx(0)
        fire_idx(1, wid + (k0 + 1) * _NW)
        run_group(0)

        drain_scatters(0)
        wait_idx(1)
        fire_idx(0, wid + (k0 + 2) * _NW)
        run_group(1)
        return c
    lax.fori_loop(0, 48, pair_body, 0)

    drain_scatters(1)
    wait_idx(0)

    @pl.when(wid < _NGROUPS - 97 * _NW)
    def _():
        fire_idx(1, wid + 97 * _NW)
    run_group(0)
    drain_scatters(0)

    @pl.when(wid < _NGROUPS - 97 * _NW)
    def _():
        wait_idx(1)
        run_group(1)
        drain_scatters(1)

    plsc.subcore_barrier()

    orow = cid * _NA + rbase
    pltpu.sync_copy(ssum_sh.at[pl.ds(rbase, _CW)],
                    ssum_out.at[pl.ds(orow, _CW)])
    pltpu.sync_copy(cnt_sh.at[pl.ds(rbase, _CW)],
                    cnt_out.at[pl.ds(orow, _CW)])


_sc_agg = functools.partial(
    pl.kernel,
    out_type=(jax.ShapeDtypeStruct((2 * _NA, 16), jnp.float32),
              jax.ShapeDtypeStruct((2 * _NA,), jnp.float32)),
    mesh=plsc.VectorSubcoreMesh(core_axis_name="c", subcore_axis_name="s"),
    compiler_params=pltpu.CompilerParams(use_tc_tiling_on_sc=False),
    scratch_types=[
        pltpu.VMEM((2, _GROUP, _CH), jnp.int32),
        pltpu.VMEM((2, _GROUP, _CH), jnp.int32),
        pltpu.VMEM((_GROUP, _CH, 16), jnp.float32),
        pltpu.VMEM((_CH,), jnp.float32),
        pltpu.VMEM_SHARED((_NA, 16), jnp.float32),
        pltpu.VMEM_SHARED((_NA,), jnp.float32),
    ] + [pltpu.SemaphoreType.DMA] * 11,
)(_sc_agg_body)


def _lin1_body(x_ref, w_ref, b_ref, o_ref):
    o_ref[...] = jnp.maximum(
        jnp.dot(x_ref[...], w_ref[...], preferred_element_type=jnp.float32)
        + b_ref[...], 0.0)


def _final_body(s0, s1, c0, c1, h_ref, wl, bl, wr, w2, b2, o_ref):
    cnt = jnp.maximum(c0[...] + c1[...], 1.0)
    aggr = (s0[...] + s1[...]) / cnt
    h2 = jnp.maximum(
        jnp.dot(aggr, wl[...], preferred_element_type=jnp.float32) + bl[...]
        + jnp.dot(h_ref[...], wr[...], preferred_element_type=jnp.float32),
        0.0)
    o_ref[...] = (jnp.dot(h2, w2[...], preferred_element_type=jnp.float32)
                  + b2[...])


@jax.jit
def kernel(x, edge_index, W1, b1, Wl, bl, Wr, W2, b2):
    eye8 = jnp.eye(8, dtype=jnp.float32)
    w1d = jnp.kron(eye8, W1.T)
    b1p = jnp.tile(b1, 8).reshape(1, 128)
    xp = jnp.pad(x, ((0, _NA - _N), (0, 0))).reshape(_NP, 128)

    hp = pl.pallas_call(
        _lin1_body,
        grid=(_GRIDP,),
        in_specs=[pl.BlockSpec((_BP, 128), lambda i: (i, 0)),
                  pl.BlockSpec((128, 128), lambda i: (0, 0)),
                  pl.BlockSpec((1, 128), lambda i: (0, 0))],
        out_specs=pl.BlockSpec((_BP, 128), lambda i: (i, 0)),
        out_shape=jax.ShapeDtypeStruct((_NP, 128), jnp.float32),
    )(xp, w1d, b1p)

    h = hp.reshape(_NA, 16)
    e3 = edge_index.reshape(2, _NCHUNKS, _CH)
    z16 = jnp.zeros((_CW, 16), jnp.float32)
    z1 = jnp.zeros((_CW,), jnp.float32)
    ssum_p, cnt_p = _sc_agg(e3, h, z16, z1)

    sp = ssum_p.reshape(2 * _NP, 128)
    cexp = jnp.broadcast_to(cnt_p.reshape(2 * _NA, 1), (2 * _NA, 16))
    cp = cexp.reshape(2 * _NP, 128)

    wld = jnp.kron(eye8, Wl.T)
    wrd = jnp.kron(eye8, Wr.T)
    w2d = jnp.kron(eye8, W2.T)
    blp = jnp.tile(bl, 8).reshape(1, 128)
    b2p = jnp.tile(b2, 8).reshape(1, 256)

    outp = pl.pallas_call(
        _final_body,
        grid=(_GRIDP,),
        in_specs=[pl.BlockSpec((_BP, 128), lambda i: (i, 0)),
                  pl.BlockSpec((_BP, 128), lambda i: (i + _GRIDP, 0)),
                  pl.BlockSpec((_BP, 128), lambda i: (i, 0)),
                  pl.BlockSpec((_BP, 128), lambda i: (i + _GRIDP, 0)),
                  pl.BlockSpec((_BP, 128), lambda i: (i, 0)),
                  pl.BlockSpec((128, 128), lambda i: (0, 0)),
                  pl.BlockSpec((1, 128), lambda i: (0, 0)),
                  pl.BlockSpec((128, 128), lambda i: (0, 0)),
                  pl.BlockSpec((128, 256), lambda i: (0, 0)),
                  pl.BlockSpec((1, 256), lambda i: (0, 0))],
        out_specs=pl.BlockSpec((_BP, 256), lambda i: (i, 0)),
        out_shape=jax.ShapeDtypeStruct((_NP, 256), jnp.float32),
    )(sp, sp, cp, cp, hp, wld, blp, wrd, w2d, b2p)
    return outp.reshape(_NA, 32)[:_N]


# speedup vs baseline: 60.8934x; 1.0743x over previous
import functools

import jax
import jax.numpy as jnp
from jax import lax
from jax.experimental import pallas as pl
from jax.experimental.pallas import tpu as pltpu
from jax.experimental.pallas import tpu_sc as plsc

_N = 100000
_E = 3200000
_CH = 128
_GROUP = 8
_NW = 32
_NGROUPS = _E // (_CH * _GROUP)
_NCHUNKS = _E // _CH
_NA = _N + 96
_CW = 6256
_NP = _NA // 8
_BP = 3128
_GRIDP = _NP // _BP


def _sc_agg_body(e3_hbm, h_hbm, z16_hbm, z1_hbm, ssum_out, cnt_out,
                 sidx, didx, rbuf, ones,
                 ssum_sh, cnt_sh,
                 g0, g1, g2, g3, g4, g5, g6, g7, ssem, csem, isem):
    cid = lax.axis_index("c")
    sid = lax.axis_index("s")
    wid = cid * 16 + sid
    gsems = (g0, g1, g2, g3, g4, g5, g6, g7)

    one16 = jnp.ones((16,), jnp.float32)
    for j in range(_CH // 16):
        ones[pl.ds(j * 16, 16)] = one16

    rbase = sid * _CW
    pltpu.sync_copy(z16_hbm, ssum_sh.at[pl.ds(rbase, _CW)])
    pltpu.sync_copy(z1_hbm, cnt_sh.at[pl.ds(rbase, _CW)])

    plsc.subcore_barrier()

    _GE = _GROUP * _CH

    def fire_idx(b, g):
        pltpu.async_copy(e3_hbm.at[0, pl.ds(g * _GE, _GE)], sidx.at[b], isem)
        pltpu.async_copy(e3_hbm.at[1, pl.ds(g * _GE, _GE)], didx.at[b], isem)

    def wait_idx(b):
        pltpu.make_async_copy(e3_hbm.at[0, pl.ds(0, _GE)], sidx.at[b],
                              isem).wait()
        pltpu.make_async_copy(e3_hbm.at[1, pl.ds(0, _GE)], didx.at[b],
                              isem).wait()

    def drain_scatters(ob):
        for j in range(_GROUP):
            dj = didx.at[ob, pl.ds(j * _CH, _CH)]
            pltpu.make_async_copy(rbuf.at[j], ssum_sh.at[dj], ssem).wait()
            pltpu.make_async_copy(ones, cnt_sh.at[dj], csem).wait()

    def run_group(b):
        gd = [pltpu.async_copy(h_hbm.at[sidx.at[b, pl.ds(j * _CH, _CH)]],
                               rbuf.at[j], gsems[j])
              for j in range(_GROUP)]
        for j in range(_GROUP):
            gd[j].wait()
            dj = didx.at[b, pl.ds(j * _CH, _CH)]
            pltpu.async_copy(rbuf.at[j], ssum_sh.at[dj], ssem, add=True)
            pltpu.async_copy(ones, cnt_sh.at[dj], csem, add=True)

    fire_idx(0, wid)

    def pair_body(k2, c):
        k0 = 2 * k2

        @pl.when(k2 > 0)
        def _():
            drain_scatters(1)
        wait_idx(0)
        fire_idx(1, wid + (k0 + 1) * _NW)
        run_group(0)

        drain_scatters(0)
        wait_idx(1)
        fire_idx(0, wid + (k0 + 2) * _NW)
        run_group(1)
        return c
    lax.fori_loop(0, 48, pair_body, 0)

    drain_scatters(1)
    wait_idx(0)

    @pl.when(wid < _NGROUPS - 97 * _NW)
    def _():
        fire_idx(1, wid + 97 * _NW)
    run_group(0)
    drain_scatters(0)

    @pl.when(wid < _NGROUPS - 97 * _NW)
    def _():
        wait_idx(1)
        run_group(1)
        drain_scatters(1)

    plsc.subcore_barrier()

    orow = cid * _NA + rbase
    pltpu.sync_copy(ssum_sh.at[pl.ds(rbase, _CW)],
                    ssum_out.at[pl.ds(orow, _CW)])
    pltpu.sync_copy(cnt_sh.at[pl.ds(rbase, _CW)],
                    cnt_out.at[pl.ds(orow, _CW)])


_sc_agg = functools.partial(
    pl.kernel,
    out_type=(jax.ShapeDtypeStruct((2 * _NA, 16), jnp.float32),
              jax.ShapeDtypeStruct((2 * _NA,), jnp.float32)),
    mesh=plsc.VectorSubcoreMesh(core_axis_name="c", subcore_axis_name="s"),
    compiler_params=pltpu.CompilerParams(use_tc_tiling_on_sc=False),
    scratch_types=[
        pltpu.VMEM((2, _GROUP * _CH), jnp.int32),
        pltpu.VMEM((2, _GROUP * _CH), jnp.int32),
        pltpu.VMEM((_GROUP, _CH, 16), jnp.float32),
        pltpu.VMEM((_CH,), jnp.float32),
        pltpu.VMEM_SHARED((_NA, 16), jnp.float32),
        pltpu.VMEM_SHARED((_NA,), jnp.float32),
    ] + [pltpu.SemaphoreType.DMA] * 11,
)(_sc_agg_body)


def _lin1_body(x_ref, w_ref, b_ref, o_ref):
    o_ref[...] = jnp.maximum(
        jnp.dot(x_ref[...], w_ref[...], preferred_element_type=jnp.float32)
        + b_ref[...], 0.0)


def _final_body(s0, s1, c0, c1, h_ref, wl, bl, wr, w2, b2, o_ref):
    cnt = jnp.maximum(c0[...] + c1[...], 1.0)
    aggr = (s0[...] + s1[...]) / cnt
    h2 = jnp.maximum(
        jnp.dot(aggr, wl[...], preferred_element_type=jnp.float32) + bl[...]
        + jnp.dot(h_ref[...], wr[...], preferred_element_type=jnp.float32),
        0.0)
    o_ref[...] = (jnp.dot(h2, w2[...], preferred_element_type=jnp.float32)
                  + b2[...])


@jax.jit
def kernel(x, edge_index, W1, b1, Wl, bl, Wr, W2, b2):
    eye8 = jnp.eye(8, dtype=jnp.float32)
    w1d = jnp.kron(eye8, W1.T)
    b1p = jnp.tile(b1, 8).reshape(1, 128)
    xp = jnp.pad(x.reshape(_N // 8, 128), ((0, (_NA - _N) // 8), (0, 0)))

    hp = pl.pallas_call(
        _lin1_body,
        grid=(_GRIDP,),
        in_specs=[pl.BlockSpec((_BP, 128), lambda i: (i, 0)),
                  pl.BlockSpec((128, 128), lambda i: (0, 0)),
                  pl.BlockSpec((1, 128), lambda i: (0, 0))],
        out_specs=pl.BlockSpec((_BP, 128), lambda i: (i, 0)),
        out_shape=jax.ShapeDtypeStruct((_NP, 128), jnp.float32),
    )(xp, w1d, b1p)

    h = hp.reshape(_NA, 16)
    e3 = edge_index
    z16 = jnp.zeros((_CW, 16), jnp.float32)
    z1 = jnp.zeros((_CW,), jnp.float32)
    ssum_p, cnt_p = _sc_agg(e3, h, z16, z1)

    sp = ssum_p.reshape(2 * _NP, 128)
    cexp = jnp.broadcast_to(cnt_p.reshape(2 * _NA, 1), (2 * _NA, 16))
    cp = cexp.reshape(2 * _NP, 128)

    wld = jnp.kron(eye8, Wl.T)
    wrd = jnp.kron(eye8, Wr.T)
    w2d = jnp.kron(eye8, W2.T)
    blp = jnp.tile(bl, 8).reshape(1, 128)
    b2p = jnp.tile(b2, 8).reshape(1, 256)

    outp = pl.pallas_call(
        _final_body,
        grid=(_GRIDP,),
        in_specs=[pl.BlockSpec((_BP, 128), lambda i: (i, 0)),
                  pl.BlockSpec((_BP, 128), lambda i: (i + _GRIDP, 0)),
                  pl.BlockSpec((_BP, 128), lambda i: (i, 0)),
                  pl.BlockSpec((_BP, 128), lambda i: (i + _GRIDP, 0)),
                  pl.BlockSpec((_BP, 128), lambda i: (i, 0)),
                  pl.BlockSpec((128, 128), lambda i: (0, 0)),
                  pl.BlockSpec((1, 128), lambda i: (0, 0)),
                  pl.BlockSpec((128, 128), lambda i: (0, 0)),
                  pl.BlockSpec((128, 256), lambda i: (0, 0)),
                  pl.BlockSpec((1, 256), lambda i: (0, 0))],
        out_specs=pl.BlockSpec((_BP, 256), lambda i: (i, 0)),
        out_shape=jax.ShapeDtypeStruct((_NP, 256), jnp.float32),
    )(sp, sp, cp, cp, hp, wld, blp, wrd, w2d, b2p)
    return outp[:_N // 8].reshape(_N, 32)
